# baseline (speedup 1.0000x reference)
import functools

import jax
import jax.numpy as jnp
from jax import lax
from jax.experimental import pallas as pl
from jax.experimental.pallas import tpu as pltpu
from jax.experimental.pallas import tpu_sc as plsc

TRAIN = 100000
BATCH = 16384
BITS = 64
LANES = 16
NC = 2
NS = 16
NW = NC * NS
CPW = BITS // NW
QCH = 4096
NQ = BATCH // QCH
UNROLL = 16

_mesh = plsc.VectorSubcoreMesh(core_axis_name="c", subcore_axis_name="s")


@functools.partial(
    pl.kernel,
    mesh=_mesh,
    compiler_params=pltpu.CompilerParams(needs_layout_passes=False),
    out_type=jax.ShapeDtypeStruct((NW, LANES), jnp.float32),
    scratch_types=[
        pltpu.VMEM((TRAIN,), jnp.float32),
        pltpu.VMEM((BATCH,), jnp.int32),
        pltpu.VMEM((2, QCH), jnp.float32),
        pltpu.VMEM((LANES,), jnp.float32),
        pltpu.SemaphoreType.DMA,
        pltpu.SemaphoreType.DMA,
        pltpu.SemaphoreType.DMA,
        pltpu.SemaphoreType.DMA,
    ],
)
def _mse_partials(xT_hbm, idx_hbm, HT_hbm, out_hbm,
                  hc_v, idx_v, xq_v, acc_v, sem_h, sem_x0, sem_x1, sem_i):
    wid = lax.axis_index("s") * NC + lax.axis_index("c")
    sem_x = (sem_x0, sem_x1)

    idx_cp = pltpu.async_copy(idx_hbm, idx_v, sem_i)
    acc = (jnp.zeros((LANES,), jnp.float32),) * UNROLL
    for k in range(CPW):
        c = wid * CPW + k
        hc_cp = pltpu.async_copy(HT_hbm.at[c], hc_v, sem_h)
        pltpu.async_copy(xT_hbm.at[c, pl.ds(0, QCH)], xq_v.at[0], sem_x[0])
        hc_cp.wait()
        if k == 0:
            idx_cp.wait()
        for q in range(NQ):
            if q + 1 < NQ:
                pltpu.async_copy(
                    xT_hbm.at[c, pl.ds((q + 1) * QCH, QCH)],
                    xq_v.at[(q + 1) % 2], sem_x[(q + 1) % 2])
            pltpu.make_async_copy(
                xT_hbm.at[c, pl.ds(0, QCH)], xq_v.at[q % 2], sem_x[q % 2]
            ).wait()

            def chunk_body(i, acc, q=q):
                new_acc = []
                for u in range(UNROLL):
                    off = (i * UNROLL + u) * LANES
                    idx16 = idx_v[pl.ds(q * QCH + off, LANES)]
                    h16 = plsc.load_gather(hc_v, [idx16])
                    x16 = xq_v[q % 2, pl.ds(off, LANES)]
                    d = x16 - h16
                    new_acc.append(acc[u] + d * d)
                return tuple(new_acc)

            acc = lax.fori_loop(0, QCH // (LANES * UNROLL), chunk_body, acc)

    total = acc[0]
    for u in range(1, UNROLL):
        total = total + acc[u]
    acc_v[...] = total
    pltpu.sync_copy(acc_v, out_hbm.at[wid])


def kernel(x, y, idx, H):
    partials = _mse_partials(x.T, idx.astype(jnp.int32), H.T)
    return jnp.sum(partials) / jnp.float32(BATCH * BITS)


# speedup vs baseline: 1.0348x; 1.0348x over previous
import functools

import jax
import jax.numpy as jnp
from jax import lax
from jax.experimental import pallas as pl
from jax.experimental.pallas import tpu as pltpu
from jax.experimental.pallas import tpu_sc as plsc

TRAIN = 100000
BATCH = 16384
BITS = 64
LANES = 16
NC = 2
NS = 16
NW = NC * NS
CPW = BITS // NW
QCH = 4096
NQ = BATCH // QCH
UNROLL = 8

_mesh = plsc.VectorSubcoreMesh(core_axis_name="c", subcore_axis_name="s")


@functools.partial(
    pl.kernel,
    mesh=_mesh,
    compiler_params=pltpu.CompilerParams(needs_layout_passes=False),
    out_type=jax.ShapeDtypeStruct((NW, LANES), jnp.float32),
    scratch_types=[
        pltpu.VMEM((TRAIN,), jnp.float32),
        pltpu.VMEM((BATCH,), jnp.int32),
        pltpu.VMEM((2, QCH), jnp.float32),
        pltpu.VMEM((LANES,), jnp.float32),
        pltpu.SemaphoreType.DMA,
        pltpu.SemaphoreType.DMA,
        pltpu.SemaphoreType.DMA,
        pltpu.SemaphoreType.DMA,
    ],
)
def _mse_partials(xT_hbm, idx_hbm, HT_hbm, out_hbm,
                  hc_v, idx_v, xq_v, acc_v, sem_h, sem_x0, sem_x1, sem_i):
    wid = lax.axis_index("s") * NC + lax.axis_index("c")
    sem_x = (sem_x0, sem_x1)

    idx_cp = pltpu.async_copy(idx_hbm, idx_v, sem_i)
    acc = (jnp.zeros((LANES,), jnp.float32),) * UNROLL
    for k in range(CPW):
        c = wid * CPW + k
        hc_cp = pltpu.async_copy(HT_hbm.at[c], hc_v, sem_h)
        pltpu.async_copy(xT_hbm.at[c, pl.ds(0, QCH)], xq_v.at[0], sem_x[0])
        hc_cp.wait()
        if k == 0:
            idx_cp.wait()
        for q in range(NQ):
            if q + 1 < NQ:
                pltpu.async_copy(
                    xT_hbm.at[c, pl.ds((q + 1) * QCH, QCH)],
                    xq_v.at[(q + 1) % 2], sem_x[(q + 1) % 2])
            pltpu.make_async_copy(
                xT_hbm.at[c, pl.ds(0, QCH)], xq_v.at[q % 2], sem_x[q % 2]
            ).wait()

            def chunk_body(i, accs, q=q):
                idx16 = idx_v[pl.ds(q * QCH + i * LANES, LANES)]
                h16 = plsc.load_gather(hc_v, [idx16])
                x16 = xq_v[q % 2, pl.ds(i * LANES, LANES)]
                d = x16 - h16
                return accs[1:] + (accs[0] + d * d,)

            acc = plsc.parallel_loop(
                0, QCH // LANES, unroll=UNROLL, carry=acc)(chunk_body)

    total = acc[0]
    for u in range(1, UNROLL):
        total = total + acc[u]
    acc_v[...] = total
    pltpu.sync_copy(acc_v, out_hbm.at[wid])


def kernel(x, y, idx, H):
    partials = _mse_partials(x.T, idx.astype(jnp.int32), H.T)
    return jnp.sum(partials) / jnp.float32(BATCH * BITS)
